# baseline (speedup 1.0000x reference)
import jax
import jax.numpy as jnp
from jax import lax
from jax.experimental import pallas as pl
from jax.experimental.pallas import tpu as pltpu
from jax.experimental.pallas import tpu_sc as plsc
import functools

T_ROWS = 1000
D = 4096
NC = 2
NS = 16
L = 16
NW = NC * NS
CHUNK = D // NW

_mesh = plsc.VectorSubcoreMesh(
    core_axis_name="c", subcore_axis_name="s", num_cores=NC, num_subcores=NS
)


@functools.partial(
    pl.kernel,
    out_type=jax.ShapeDtypeStruct((D,), jnp.float32),
    mesh=_mesh,
    scratch_types=[
        pltpu.VMEM((L,), jnp.float32),
        pltpu.VMEM((2, CHUNK), jnp.float32),
        pltpu.VMEM((CHUNK,), jnp.float32),
    ],
    compiler_params=pltpu.CompilerParams(use_tc_tiling_on_sc=False, needs_layout_passes=False),
)
def _lerp_lookup(t_hbm, emb_hbm, out_hbm, t_v, rows_v, out_v):
    wid = lax.axis_index("s") * NC + lax.axis_index("c")
    col = wid * CHUNK

    pltpu.sync_copy(t_hbm, t_v)
    tv = t_v[...]
    time = (tv + 1.0) * (0.5 * (T_ROWS - 1))
    t0 = time.astype(jnp.int32)
    alpha = time - t0.astype(jnp.float32)
    t0c = jnp.minimum(t0, T_ROWS - 2)
    alpha = jnp.where(t0 > T_ROWS - 2, 1.0, alpha)
    t0s = jnp.max(t0c)

    pltpu.sync_copy(emb_hbm.at[pl.ds(t0s, 2), pl.ds(col, CHUNK)], rows_v)

    for j in range(CHUNK // L):
        lo = rows_v[0, pl.ds(j * L, L)]
        hi = rows_v[1, pl.ds(j * L, L)]
        out_v[pl.ds(j * L, L)] = lo + alpha * (hi - lo)

    pltpu.sync_copy(out_v, out_hbm.at[pl.ds(col, CHUNK)])


def kernel(t, time_emb):
    t_vec = jnp.full((L,), t, dtype=jnp.float32)
    return _lerp_lookup(t_vec, time_emb)


# speedup vs baseline: 1.7012x; 1.7012x over previous
import jax
import jax.numpy as jnp
from jax import lax
from jax.experimental import pallas as pl
from jax.experimental.pallas import tpu as pltpu
from jax.experimental.pallas import tpu_sc as plsc
import functools

T_ROWS = 1000
D = 4096
NC = 2
NS = 16
L = 16
NW = NC * NS
CHUNK = D // NW
WIN = 16

_mesh = plsc.VectorSubcoreMesh(
    core_axis_name="c", subcore_axis_name="s", num_cores=NC, num_subcores=NS
)


@functools.partial(
    pl.kernel,
    out_type=jax.ShapeDtypeStruct((D,), jnp.float32),
    mesh=_mesh,
    scratch_types=[
        pltpu.VMEM((L,), jnp.float32),
        pltpu.VMEM((WIN, CHUNK), jnp.float32),
        pltpu.VMEM((CHUNK,), jnp.float32),
    ],
    compiler_params=pltpu.CompilerParams(needs_layout_passes=False),
)
def _lerp_lookup(t_hbm, emb_hbm, out_hbm, t_v, rows_v, out_v):
    wid = lax.axis_index("s") * NC + lax.axis_index("c")
    col = wid * CHUNK

    pltpu.sync_copy(t_hbm, t_v)
    tv = t_v[...]
    time = (tv + 1.0) * (0.5 * (T_ROWS - 1))
    t0 = time.astype(jnp.int32)
    alpha = time - t0.astype(jnp.float32)
    t0c = jnp.minimum(t0, T_ROWS - 2)
    alpha = jnp.where(t0 > T_ROWS - 2, jnp.float32(1.0), alpha)
    base = jnp.minimum(t0c & ~7, T_ROWS - WIN)
    off = t0c - base
    base_s = pl.multiple_of(jnp.max(base), 8)

    pltpu.sync_copy(emb_hbm.at[pl.ds(base_s, WIN), pl.ds(col, CHUNK)], rows_v)

    zero = jnp.zeros((L,), jnp.float32)
    w = [
        jnp.where(off == r, 1.0 - alpha, zero)
        + jnp.where(off + 1 == r, alpha, zero)
        for r in range(WIN)
    ]

    for j in range(CHUNK // L):
        acc = rows_v[0, pl.ds(j * L, L)] * w[0]
        for r in range(1, WIN):
            acc = acc + rows_v[r, pl.ds(j * L, L)] * w[r]
        out_v[pl.ds(j * L, L)] = acc

    pltpu.sync_copy(out_v, out_hbm.at[pl.ds(col, CHUNK)])


def kernel(t, time_emb):
    t_vec = jnp.full((L,), t, dtype=jnp.float32)
    return _lerp_lookup(t_vec, time_emb)


# speedup vs baseline: 1.7064x; 1.0031x over previous
import jax
import jax.numpy as jnp
from jax import lax
from jax.experimental import pallas as pl
from jax.experimental.pallas import tpu as pltpu
from jax.experimental.pallas import tpu_sc as plsc
import functools

T_ROWS = 1000
D = 4096
NC = 2
NS = 16
L = 16
NW = NC * NS
CHUNK = D // NW
WIN = 16

_mesh = plsc.VectorSubcoreMesh(
    core_axis_name="c", subcore_axis_name="s", num_cores=NC, num_subcores=NS
)


@functools.partial(
    pl.kernel,
    out_type=jax.ShapeDtypeStruct((D,), jnp.float32),
    mesh=_mesh,
    scratch_types=[
        pltpu.VMEM((L,), jnp.float32),
        pltpu.VMEM((WIN, CHUNK), jnp.float32),
        pltpu.VMEM((CHUNK,), jnp.float32),
    ],
    compiler_params=pltpu.CompilerParams(needs_layout_passes=False),
)
def _lerp_lookup(t_hbm, emb_hbm, out_hbm, t_v, rows_v, out_v):
    wid = lax.axis_index("s") * NC + lax.axis_index("c")
    col = wid * CHUNK

    pltpu.sync_copy(t_hbm, t_v)
    tv = t_v[...]
    time = (tv + 1.0) * (0.5 * (T_ROWS - 1))
    t0 = time.astype(jnp.int32)
    alpha = time - t0.astype(jnp.float32)
    t0c = jnp.minimum(t0, T_ROWS - 2)
    alpha = jnp.where(t0 > T_ROWS - 2, jnp.float32(1.0), alpha)
    base = jnp.minimum(t0c & ~7, T_ROWS - WIN)
    off = t0c - base
    base_s = pl.multiple_of(jnp.max(base), 8)

    pltpu.sync_copy(emb_hbm.at[pl.ds(base_s, WIN), pl.ds(col, CHUNK)], rows_v)

    lanes = jax.lax.iota(jnp.int32, L)
    for j in range(CHUNK // L):
        cid = lanes + (j * L)
        lo = plsc.load_gather(rows_v, [off, cid])
        hi = plsc.load_gather(rows_v, [off + 1, cid])
        out_v[pl.ds(j * L, L)] = lo + alpha * (hi - lo)

    pltpu.sync_copy(out_v, out_hbm.at[pl.ds(col, CHUNK)])


def kernel(t, time_emb):
    t_vec = jnp.full((L,), t, dtype=jnp.float32)
    return _lerp_lookup(t_vec, time_emb)


# speedup vs baseline: 1.7140x; 1.0045x over previous
import jax
import jax.numpy as jnp
from jax import lax
from jax.experimental import pallas as pl
from jax.experimental.pallas import tpu as pltpu
from jax.experimental.pallas import tpu_sc as plsc
import functools

T_ROWS = 1000
D = 4096
NC = 2
NS = 16
L = 16
NW = NC * NS
CHUNK = D // NW
WIN = 16

_mesh = plsc.VectorSubcoreMesh(
    core_axis_name="c", subcore_axis_name="s", num_cores=NC, num_subcores=NS
)


@functools.partial(
    pl.kernel,
    out_type=jax.ShapeDtypeStruct((D,), jnp.float32),
    mesh=_mesh,
    scratch_types=[
        pltpu.VMEM((L,), jnp.float32),
        pltpu.VMEM((WIN, CHUNK), jnp.float32),
        pltpu.VMEM((CHUNK,), jnp.float32),
    ],
    compiler_params=pltpu.CompilerParams(
        needs_layout_passes=False, skip_device_barrier=True
    ),
)
def _lerp_lookup(t_hbm, emb_hbm, out_hbm, t_v, rows_v, out_v):
    wid = lax.axis_index("s") * NC + lax.axis_index("c")
    col = wid * CHUNK

    pltpu.sync_copy(t_hbm, t_v)
    tv = t_v[...]
    time = (tv + 1.0) * (0.5 * (T_ROWS - 1))
    t0 = time.astype(jnp.int32)
    alpha = time - t0.astype(jnp.float32)
    t0c = jnp.minimum(t0, T_ROWS - 2)
    alpha = jnp.where(t0 > T_ROWS - 2, jnp.float32(1.0), alpha)
    base = jnp.minimum(t0c & ~7, T_ROWS - WIN)
    off = t0c - base
    base_s = pl.multiple_of(jnp.max(base), 8)

    pltpu.sync_copy(emb_hbm.at[pl.ds(base_s, WIN), pl.ds(col, CHUNK)], rows_v)

    lanes = jax.lax.iota(jnp.int32, L)
    for j in range(CHUNK // L):
        cid = lanes + (j * L)
        lo = plsc.load_gather(rows_v, [off, cid])
        hi = plsc.load_gather(rows_v, [off + 1, cid])
        out_v[pl.ds(j * L, L)] = lo + alpha * (hi - lo)

    pltpu.sync_copy(out_v, out_hbm.at[pl.ds(col, CHUNK)])


def kernel(t, time_emb):
    t_vec = jnp.full((L,), t, dtype=jnp.float32)
    return _lerp_lookup(t_vec, time_emb)


# speedup vs baseline: 1.8546x; 1.0821x over previous
import jax
import jax.numpy as jnp
from jax import lax
from jax.experimental import pallas as pl
from jax.experimental.pallas import tpu as pltpu
from jax.experimental.pallas import tpu_sc as plsc
import functools

T_ROWS = 1000
D = 4096
NC = 1
NS = 16
L = 16
NW = NC * NS
CHUNK = D // NW
WIN = 16

_mesh = plsc.VectorSubcoreMesh(
    core_axis_name="c", subcore_axis_name="s", num_cores=NC, num_subcores=NS
)


@functools.partial(
    pl.kernel,
    out_type=jax.ShapeDtypeStruct((D,), jnp.float32),
    mesh=_mesh,
    scratch_types=[
        pltpu.VMEM((L,), jnp.float32),
        pltpu.VMEM((WIN, CHUNK), jnp.float32),
        pltpu.VMEM((CHUNK,), jnp.float32),
    ],
    compiler_params=pltpu.CompilerParams(
        needs_layout_passes=False, skip_device_barrier=True
    ),
)
def _lerp_lookup(t_hbm, emb_hbm, out_hbm, t_v, rows_v, out_v):
    wid = lax.axis_index("s") * NC + lax.axis_index("c")
    col = wid * CHUNK

    pltpu.sync_copy(t_hbm, t_v)
    tv = t_v[...]
    time = (tv + 1.0) * (0.5 * (T_ROWS - 1))
    t0 = time.astype(jnp.int32)
    alpha = time - t0.astype(jnp.float32)
    t0c = jnp.minimum(t0, T_ROWS - 2)
    alpha = jnp.where(t0 > T_ROWS - 2, jnp.float32(1.0), alpha)
    base = jnp.minimum(t0c & ~7, T_ROWS - WIN)
    off = t0c - base
    base_s = pl.multiple_of(jnp.max(base), 8)

    pltpu.sync_copy(emb_hbm.at[pl.ds(base_s, WIN), pl.ds(col, CHUNK)], rows_v)

    lanes = jax.lax.iota(jnp.int32, L)
    for j in range(CHUNK // L):
        cid = lanes + (j * L)
        lo = plsc.load_gather(rows_v, [off, cid])
        hi = plsc.load_gather(rows_v, [off + 1, cid])
        out_v[pl.ds(j * L, L)] = lo + alpha * (hi - lo)

    pltpu.sync_copy(out_v, out_hbm.at[pl.ds(col, CHUNK)])


def kernel(t, time_emb):
    t_vec = jnp.full((L,), t, dtype=jnp.float32)
    return _lerp_lookup(t_vec, time_emb)


# speedup vs baseline: 1.8776x; 1.0124x over previous
import jax
import jax.numpy as jnp
from jax import lax
from jax.experimental import pallas as pl
from jax.experimental.pallas import tpu as pltpu
from jax.experimental.pallas import tpu_sc as plsc
import functools

T_ROWS = 1000
D = 4096
NC = 1
NS = 16
L = 16
NW = NC * NS
CHUNK = D // NW
WIN = 16

_mesh = plsc.VectorSubcoreMesh(
    core_axis_name="c", subcore_axis_name="s", num_cores=NC, num_subcores=NS
)


@functools.partial(
    pl.kernel,
    out_type=jax.ShapeDtypeStruct((D,), jnp.float32),
    mesh=_mesh,
    scratch_types=[
        pltpu.VMEM((L,), jnp.float32),
        pltpu.VMEM((WIN, CHUNK), jnp.float32),
        pltpu.VMEM((CHUNK,), jnp.float32),
    ],
    compiler_params=pltpu.CompilerParams(
        needs_layout_passes=False, skip_device_barrier=True
    ),
)
def _lerp_lookup(t_hbm, emb_hbm, out_hbm, t_v, rows_v, out_v):
    wid = lax.axis_index("s") * NC + lax.axis_index("c")
    col = wid * CHUNK

    pltpu.sync_copy(t_hbm, t_v.at[pl.ds(0, 1)])
    zeros = jnp.zeros((L,), jnp.int32)
    tv = plsc.load_gather(t_v, [zeros])
    time = (tv + 1.0) * (0.5 * (T_ROWS - 1))
    t0 = time.astype(jnp.int32)
    alpha = time - t0.astype(jnp.float32)
    t0c = jnp.minimum(t0, T_ROWS - 2)
    alpha = jnp.where(t0 > T_ROWS - 2, jnp.float32(1.0), alpha)
    base = jnp.minimum(t0c & ~7, T_ROWS - WIN)
    off = t0c - base
    base_s = pl.multiple_of(jnp.max(base), 8)

    pltpu.sync_copy(emb_hbm.at[pl.ds(base_s, WIN), pl.ds(col, CHUNK)], rows_v)

    lanes = jax.lax.iota(jnp.int32, L)
    for j in range(CHUNK // L):
        cid = lanes + (j * L)
        lo = plsc.load_gather(rows_v, [off, cid])
        hi = plsc.load_gather(rows_v, [off + 1, cid])
        out_v[pl.ds(j * L, L)] = lo + alpha * (hi - lo)

    pltpu.sync_copy(out_v, out_hbm.at[pl.ds(col, CHUNK)])


def kernel(t, time_emb):
    return _lerp_lookup(t.reshape(1), time_emb)
